# baseline (speedup 1.0000x reference)
import functools

import jax
import jax.numpy as jnp
from jax import lax
from jax.experimental import pallas as pl
from jax.experimental.pallas import tpu as pltpu
from jax.experimental.pallas import tpu_sc as plsc

B = 4096
T = 200
C = 64
NC = 2
NS = 16
NW = NC * NS
ROWS_PER_W = B // NW
HALF = T // 2
CHUNKS = 2 * ROWS_PER_W
NBUF = 8
LOOKAHEAD = 4
MACROS = CHUNKS // NBUF
LANES = 16
VECS_PER_ROW = C // LANES


def _body(idx_hbm, tok_hbm, pos_hbm, out_hbm, idx_v, pos_v, *rest):
    bufs = rest[:NBUF]
    gsem = rest[NBUF:2 * NBUF]
    ssem = rest[2 * NBUF:3 * NBUF]
    cid = lax.axis_index("c")
    sid = lax.axis_index("s")
    w = sid * NC + cid

    pltpu.sync_copy(idx_hbm.at[w], idx_v)
    pltpu.sync_copy(pos_hbm, pos_v)

    for b in range(LOOKAHEAD):
        pltpu.async_copy(tok_hbm.at[idx_v.at[b]], bufs[b], gsem[b])

    def macro(m, carry):
        for b in range(NBUF):
            k = NBUF * m + b
            r = w * ROWS_PER_W + 4 * m + b // 2
            pb = (b % 2) * HALF

            pltpu.make_async_copy(
                tok_hbm.at[idx_v.at[k]], bufs[b], gsem[b]).wait()

            def add_i(i, c2, _b=b, _pb=pb):
                for c in range(VECS_PER_ROW):
                    sl = pl.ds(c * LANES, LANES)
                    plsc.addupdate(bufs[_b].at[i, sl], pos_v[_pb + i, sl])
                return c2

            lax.fori_loop(0, HALF, add_i, 0)

            pltpu.async_copy(bufs[b], out_hbm.at[r, pl.ds(pb, HALF)], ssem[b])

            b2 = (b + LOOKAHEAD) % NBUF

            def wait_prev_store(_b2=b2):
                pltpu.make_async_copy(
                    bufs[_b2], out_hbm.at[r, pl.ds(pb, HALF)], ssem[_b2]).wait()

            def issue_gather(_b2=b2, _k=k):
                pltpu.async_copy(
                    tok_hbm.at[idx_v.at[_k + LOOKAHEAD]], bufs[_b2], gsem[_b2])

            if b < LOOKAHEAD:
                @pl.when(m > 0)
                def _():
                    wait_prev_store()

                issue_gather()
            else:
                @pl.when(m < MACROS - 1)
                def _():
                    wait_prev_store()
                    issue_gather()
        return carry

    lax.fori_loop(0, MACROS, macro, 0)

    for b in range(NBUF):
        pltpu.make_async_copy(
            bufs[b], out_hbm.at[0, pl.ds(0, HALF)], ssem[b]).wait()


def _run(idx32, tok, pos):
    mesh = plsc.VectorSubcoreMesh(core_axis_name="c", subcore_axis_name="s")
    k = functools.partial(
        pl.kernel,
        mesh=mesh,
        out_type=jax.ShapeDtypeStruct((B, T, C), jnp.float32),
        scratch_types=(
            [pltpu.VMEM((CHUNKS, HALF), jnp.int32),
             pltpu.VMEM((T, C), jnp.float32)]
            + [pltpu.VMEM((HALF, C), jnp.float32) for _ in range(NBUF)]
            + [pltpu.SemaphoreType.DMA for _ in range(2 * NBUF)]
        ),
        compiler_params=pltpu.CompilerParams(use_tc_tiling_on_sc=False),
    )(_body)
    return k(idx32, tok, pos)


def kernel(idx, token_embedding_table, position_embedding_table):
    idx32 = idx.astype(jnp.int32).reshape(NW, CHUNKS, HALF)
    return _run(idx32, token_embedding_table, position_embedding_table)
